# speedup vs baseline: 1.0806x; 1.0806x over previous
import jax
import jax.numpy as jnp
from jax.experimental import pallas as pl
from jax.experimental.pallas import tpu as pltpu

N = 10000
D_IN = 128
D_G = 128
D_H = 256
D_L = 64


def _mm_bias_body(x_ref, w_ref, b_ref, o_ref):
    o_ref[...] = (
        jnp.dot(x_ref[...], w_ref[...], preferred_element_type=jnp.float32)
        + b_ref[...]
    )


def _mm_bias(x, w, b):
    n, k = x.shape
    m = w.shape[1]
    nb = 10
    blk = n // nb
    return pl.pallas_call(
        _mm_bias_body,
        out_shape=jax.ShapeDtypeStruct((n, m), jnp.float32),
        grid=(nb,),
        in_specs=[
            pl.BlockSpec((blk, k), lambda i: (i, 0)),
            pl.BlockSpec((k, m), lambda i: (0, 0)),
            pl.BlockSpec((1, m), lambda i: (0, 0)),
        ],
        out_specs=pl.BlockSpec((blk, m), lambda i: (i, 0)),
    )(x, w, b.reshape(1, m))


def _batchnorm(h, gamma, beta, eps=1e-5):
    mean = jnp.mean(h, axis=0)
    var = jnp.var(h, axis=0)
    return (h - mean) / jnp.sqrt(var + eps) * gamma + beta


def kernel(x, edge_index, edge_weight, Wg, bg, gamma0, beta0, W1, b1,
           gamma1, beta1, W21, b21, W22, b22):
    h = _mm_bias(x, Wg, bg)
    dst = edge_index[0]
    src = edge_index[1]
    msgs = edge_weight[:, None] * jnp.take(h, src, axis=0)
    agg = jax.ops.segment_sum(msgs, dst, num_segments=x.shape[0])
    h0 = jax.nn.softplus(_batchnorm(agg, gamma0, beta0))
    h1 = jax.nn.softplus(_batchnorm(h0 @ W1 + b1, gamma1, beta1))
    mu = h1 @ W21 + b21
    logvar = h1 @ W22 + b22
    return (mu, logvar)


# speedup vs baseline: 8.2702x; 7.6531x over previous
import functools

import jax
import jax.numpy as jnp
from jax import lax
from jax.experimental import pallas as pl
from jax.experimental.pallas import tpu as pltpu
from jax.experimental.pallas import tpu_sc as plsc

N = 10000
D_IN = 128
D_G = 128
D_H = 256
D_L = 64
E_TOT = 320000

_NC = 2
_NS = 16
_NW = _NC * _NS
_CH = 80
_RPT = E_TOT // _NW // _CH
_NSEC = 5
_SEC = _RPT // _NSEC
_EPS = 1e-5


def _mm_bias_body(x_ref, w_ref, b_ref, o_ref):
    o_ref[...] = (
        jnp.dot(x_ref[...], w_ref[...], preferred_element_type=jnp.float32)
        + b_ref[...]
    )


def _mm_bias(x, w, b):
    n, k = x.shape
    m = w.shape[1]
    nb = 10
    blk = n // nb
    return pl.pallas_call(
        _mm_bias_body,
        out_shape=jax.ShapeDtypeStruct((n, m), jnp.float32),
        grid=(nb,),
        in_specs=[
            pl.BlockSpec((blk, k), lambda i: (i, 0)),
            pl.BlockSpec((k, m), lambda i: (0, 0)),
            pl.BlockSpec((1, m), lambda i: (0, 0)),
        ],
        out_specs=pl.BlockSpec((blk, m), lambda i: (i, 0)),
    )(x, w, b.reshape(1, m))


def _sc_agg(h, src3, dst3, w3, zeros_nd):
    mesh = plsc.VectorSubcoreMesh(core_axis_name="c", subcore_axis_name="s")

    @functools.partial(
        pl.kernel,
        out_type=jax.ShapeDtypeStruct((_NC, N, D_G), jnp.float32),
        mesh=mesh,
        scratch_types=[
            pltpu.VMEM((_SEC, _CH), jnp.int32),
            pltpu.VMEM((_SEC, _CH), jnp.int32),
            pltpu.VMEM((_SEC, _CH), jnp.float32),
            pltpu.VMEM((_CH, D_G), jnp.float32),
            pltpu.VMEM((_CH, D_G), jnp.float32),
            pltpu.VMEM_SHARED((N, D_G), jnp.float32),
            pltpu.SemaphoreType.DMA,
            pltpu.SemaphoreType.DMA,
        ],
    )
    def k(h_hbm, src_hbm, dst_hbm, w_hbm, z_hbm, out_hbm,
          src_v, dst_v, w_v, rows0, rows1, acc, gsem0, gsem1):
        c = lax.axis_index("c")
        s = lax.axis_index("s")
        wid = c * _NS + s

        zb = s * 624
        pltpu.sync_copy(z_hbm.at[pl.ds(zb, 624)], acc.at[pl.ds(zb, 624)])

        @pl.when(s == _NS - 1)
        def _():
            pltpu.sync_copy(z_hbm.at[pl.ds(9984, 16)],
                            acc.at[pl.ds(9984, 16)])

        plsc.subcore_barrier()

        def start_gather(g, buf, sem):
            pltpu.async_copy(h_hbm.at[src_v.at[g]], buf, sem)

        def wait_gather(buf, sem):
            pltpu.make_async_copy(h_hbm.at[src_v.at[0]], buf, sem).wait()

        def process(g, buf):
            def egroup(g16, _):
                w16 = w_v[g, pl.ds(g16 * 16, 16)]
                for l in range(16):
                    wv = jnp.full((16,), w16[l], dtype=jnp.float32)
                    r = g16 * 16 + l
                    for jj in range(8):
                        sl = pl.ds(jj * 16, 16)
                        buf[r, sl] = buf[r, sl] * wv
                return 0
            lax.fori_loop(0, _CH // 16, egroup, 0)
            pltpu.sync_copy(buf, acc.at[dst_v.at[g]], add=True)

        def section(sec, _):
            pltpu.sync_copy(src_hbm.at[wid, sec], src_v)
            pltpu.sync_copy(dst_hbm.at[wid, sec], dst_v)
            pltpu.sync_copy(w_hbm.at[wid, sec], w_v)

            start_gather(0, rows0, gsem0)

            def body(k2, _):
                g0 = 2 * k2
                start_gather(g0 + 1, rows1, gsem1)
                wait_gather(rows0, gsem0)
                process(g0, rows0)
                start_gather(g0 + 2, rows0, gsem0)
                wait_gather(rows1, gsem1)
                process(g0 + 1, rows1)
                return 0

            lax.fori_loop(0, (_SEC - 1) // 2, body, 0)
            wait_gather(rows0, gsem0)
            process(_SEC - 1, rows0)
            return 0

        lax.fori_loop(0, _NSEC, section, 0)

        plsc.subcore_barrier()
        pltpu.sync_copy(acc.at[pl.ds(zb, 624)],
                        out_hbm.at[c].at[pl.ds(zb, 624)])

        @pl.when(s == _NS - 1)
        def _():
            pltpu.sync_copy(acc.at[pl.ds(9984, 16)],
                            out_hbm.at[c].at[pl.ds(9984, 16)])

    return k(h, src3, dst3, w3, zeros_nd)


def _stats_body(p_ref, agg_ref, st_ref, acc_ref):
    i = pl.program_id(0)
    a = p_ref[0] + p_ref[1]
    agg_ref[...] = a
    st = jnp.concatenate(
        [jnp.sum(a, axis=0, keepdims=True),
         jnp.sum(a * a, axis=0, keepdims=True)], axis=0)

    @pl.when(i == 0)
    def _():
        acc_ref[...] = st

    @pl.when(i != 0)
    def _():
        acc_ref[...] = acc_ref[...] + st

    @pl.when(i == pl.num_programs(0) - 1)
    def _():
        st_ref[...] = acc_ref[...]


def _stats(parts):
    nb = 10
    blk = N // nb
    return pl.pallas_call(
        _stats_body,
        out_shape=[
            jax.ShapeDtypeStruct((N, D_G), jnp.float32),
            jax.ShapeDtypeStruct((2, D_G), jnp.float32),
        ],
        grid=(nb,),
        in_specs=[pl.BlockSpec((_NC, blk, D_G), lambda i: (0, i, 0))],
        out_specs=[
            pl.BlockSpec((blk, D_G), lambda i: (i, 0)),
            pl.BlockSpec((2, D_G), lambda i: (0, 0)),
        ],
        scratch_shapes=[pltpu.VMEM((2, D_G), jnp.float32)],
    )(parts)


def _mid_body(agg_ref, st_ref, g0_ref, b0_ref, w1_ref, b1_ref,
              u_ref, st1_ref, acc_ref):
    i = pl.program_id(0)
    st = st_ref[...]
    mean = st[0:1] * (1.0 / N)
    var = st[1:2] * (1.0 / N) - mean * mean
    inv = lax.rsqrt(var + _EPS)
    h0 = jax.nn.softplus(
        (agg_ref[...] - mean) * inv * g0_ref[...] + b0_ref[...])
    u = jnp.dot(h0, w1_ref[...], preferred_element_type=jnp.float32) + b1_ref[...]
    u_ref[...] = u
    st1 = jnp.concatenate(
        [jnp.sum(u, axis=0, keepdims=True),
         jnp.sum(u * u, axis=0, keepdims=True)], axis=0)

    @pl.when(i == 0)
    def _():
        acc_ref[...] = st1

    @pl.when(i != 0)
    def _():
        acc_ref[...] = acc_ref[...] + st1

    @pl.when(i == pl.num_programs(0) - 1)
    def _():
        st1_ref[...] = acc_ref[...]


def _mid(agg, st0, gamma0, beta0, W1, b1):
    nb = 10
    blk = N // nb
    return pl.pallas_call(
        _mid_body,
        out_shape=[
            jax.ShapeDtypeStruct((N, D_H), jnp.float32),
            jax.ShapeDtypeStruct((2, D_H), jnp.float32),
        ],
        grid=(nb,),
        in_specs=[
            pl.BlockSpec((blk, D_G), lambda i: (i, 0)),
            pl.BlockSpec((2, D_G), lambda i: (0, 0)),
            pl.BlockSpec((1, D_G), lambda i: (0, 0)),
            pl.BlockSpec((1, D_G), lambda i: (0, 0)),
            pl.BlockSpec((D_G, D_H), lambda i: (0, 0)),
            pl.BlockSpec((1, D_H), lambda i: (0, 0)),
        ],
        out_specs=[
            pl.BlockSpec((blk, D_H), lambda i: (i, 0)),
            pl.BlockSpec((2, D_H), lambda i: (0, 0)),
        ],
        scratch_shapes=[pltpu.VMEM((2, D_H), jnp.float32)],
    )(agg, st0, gamma0.reshape(1, -1), beta0.reshape(1, -1), W1,
      b1.reshape(1, -1))


def _final_body(u_ref, st_ref, g1_ref, b1_ref, w21_ref, b21_ref,
                w22_ref, b22_ref, mu_ref, lv_ref):
    st = st_ref[...]
    mean = st[0:1] * (1.0 / N)
    var = st[1:2] * (1.0 / N) - mean * mean
    inv = lax.rsqrt(var + _EPS)
    h1 = jax.nn.softplus(
        (u_ref[...] - mean) * inv * g1_ref[...] + b1_ref[...])
    mu_ref[...] = (
        jnp.dot(h1, w21_ref[...], preferred_element_type=jnp.float32)
        + b21_ref[...])
    lv_ref[...] = (
        jnp.dot(h1, w22_ref[...], preferred_element_type=jnp.float32)
        + b22_ref[...])


def _final(u, st1, gamma1, beta1, W21, b21, W22, b22):
    nb = 10
    blk = N // nb
    return pl.pallas_call(
        _final_body,
        out_shape=[
            jax.ShapeDtypeStruct((N, D_L), jnp.float32),
            jax.ShapeDtypeStruct((N, D_L), jnp.float32),
        ],
        grid=(nb,),
        in_specs=[
            pl.BlockSpec((blk, D_H), lambda i: (i, 0)),
            pl.BlockSpec((2, D_H), lambda i: (0, 0)),
            pl.BlockSpec((1, D_H), lambda i: (0, 0)),
            pl.BlockSpec((1, D_H), lambda i: (0, 0)),
            pl.BlockSpec((D_H, D_L), lambda i: (0, 0)),
            pl.BlockSpec((1, D_L), lambda i: (0, 0)),
            pl.BlockSpec((D_H, D_L), lambda i: (0, 0)),
            pl.BlockSpec((1, D_L), lambda i: (0, 0)),
        ],
        out_specs=[
            pl.BlockSpec((blk, D_L), lambda i: (i, 0)),
            pl.BlockSpec((blk, D_L), lambda i: (i, 0)),
        ],
    )(u, st1, gamma1.reshape(1, -1), beta1.reshape(1, -1),
      W21, b21.reshape(1, -1), W22, b22.reshape(1, -1))


def kernel(x, edge_index, edge_weight, Wg, bg, gamma0, beta0, W1, b1,
           gamma1, beta1, W21, b21, W22, b22):
    h = _mm_bias(x, Wg, bg)
    dst3 = edge_index[0].reshape(_NW, _NSEC, _SEC, _CH)
    src3 = edge_index[1].reshape(_NW, _NSEC, _SEC, _CH)
    w3 = edge_weight.reshape(_NW, _NSEC, _SEC, _CH)
    zeros_nd = jnp.zeros((N, D_G), jnp.float32)
    parts = _sc_agg(h, src3, dst3, w3, zeros_nd)
    agg, st0 = _stats(parts)
    u, st1 = _mid(agg, st0, gamma0, beta0, W1, b1)
    mu, logvar = _final(u, st1, gamma1, beta1, W21, b21, W22, b22)
    return (mu, logvar)
